# baseline (speedup 1.0000x reference)
import jax
import jax.numpy as jnp
from jax import lax
from jax.experimental import pallas as pl
from jax.experimental.pallas import tpu as pltpu
from jax.experimental.pallas import tpu_sc as plsc

_D = 64
_SCALE = 8.0
_W = 128
_L = 16


def _emb_pipeline(table_hbm, idx_hbm, out_hbm, rows_v, st_v, *, num_windows,
                  n_bh):
    def body(i_vmem, o_vmem):
        pltpu.sync_copy(table_hbm.at[i_vmem.at[0]], rows_v)

        lanes = lax.iota(jnp.int32, _L)

        @pl.loop(0, _W)
        def _(b):
            for c in range(_D // _L):
                st_v[c, b, pl.ds(0, _L)] = rows_v[b, pl.ds(c * _L, _L)]

        @pl.loop(0, _D // _L)
        def _(c):
            c_vec = jnp.full((_L,), c, jnp.int32)
            for j in range(_L):
                j_vec = jnp.full((_L,), j, jnp.int32)
                dh = 2 * c + (j // 8)
                dl = j % 8
                for bg in range(_W // _L):
                    b_vec = bg * _L + lanes
                    vals = plsc.load_gather(st_v, [c_vec, b_vec, j_vec])
                    o_vmem[0, dh, 0, dl, pl.ds(bg * _L, _L)] = vals * _SCALE

    pltpu.emit_pipeline(
        body,
        grid=(num_windows,),
        in_specs=[pl.BlockSpec((1, _W), index_map=lambda w: (0, w))],
        out_specs=[
            pl.BlockSpec(
                (1, 8, 1, 8, _W),
                index_map=lambda w: (w // n_bh, 0, w % n_bh, 0, 0),
            )
        ],
        core_axis_name=("c", "s"),
        dimension_semantics=(pltpu.PARALLEL,),
    )(idx_hbm, out_hbm)


def kernel(inputs, table):
    batch, seq = inputs.shape
    n = batch * seq
    n_bh = batch // _W
    idx = inputs.astype(jnp.int32).T.reshape(1, n)
    num_windows = n // _W

    @pl.kernel(
        out_type=jax.ShapeDtypeStruct((seq, 8, n_bh, 8, _W), table.dtype),
        mesh=plsc.VectorSubcoreMesh(core_axis_name="c", subcore_axis_name="s"),
        compiler_params=pltpu.CompilerParams(
            use_tc_tiling_on_sc=False, needs_layout_passes=False),
        scratch_types=[
            pltpu.VMEM((_W, _D), jnp.float32),
            pltpu.VMEM((_D // _L, _W, 17), jnp.float32),
        ],
    )
    def emb(table_hbm, idx_hbm, out_hbm, rows_v, st_v):
        _emb_pipeline(table_hbm, idx_hbm, out_hbm, rows_v, st_v,
                      num_windows=num_windows, n_bh=n_bh)

    out5d = emb(table, idx)
    return out5d.transpose(2, 4, 0, 1, 3).reshape(batch, seq, _D)


# speedup vs baseline: 1.9893x; 1.9893x over previous
import jax
import jax.numpy as jnp
from jax import lax
from jax.experimental import pallas as pl
from jax.experimental.pallas import tpu as pltpu
from jax.experimental.pallas import tpu_sc as plsc

_D = 64
_SCALE = 8.0
_W = 128
_L = 16


def _emb_pipeline(table_hbm, idx_hbm, out_hbm, rows_v, st_v, *, num_windows,
                  n_bh):
    def body(i_vmem, o_vmem):
        pltpu.sync_copy(table_hbm.at[i_vmem.at[0]], rows_v)

        lanes = lax.iota(jnp.int32, _L)

        @plsc.parallel_loop(0, _W, unroll=4)
        def _(b):
            for c in range(_D // _L):
                st_v[c, b, pl.ds(0, _L)] = rows_v[b, pl.ds(c * _L, _L)]

        @plsc.parallel_loop(0, _D, unroll=4)
        def _(d):
            c = d // _L
            j = d % _L
            c_vec = jnp.full((_L,), c, jnp.int32)
            j_vec = jnp.full((_L,), j, jnp.int32)
            dh = d // 8
            dl = d % 8
            for bg in range(_W // _L):
                b_vec = bg * _L + lanes
                vals = plsc.load_gather(st_v, [c_vec, b_vec, j_vec])
                o_vmem[0, dh, 0, dl, pl.ds(bg * _L, _L)] = vals * _SCALE

    pltpu.emit_pipeline(
        body,
        grid=(num_windows,),
        in_specs=[pl.BlockSpec((1, _W), index_map=lambda w: (0, w))],
        out_specs=[
            pl.BlockSpec(
                (1, 8, 1, 8, _W),
                index_map=lambda w: (w // n_bh, 0, w % n_bh, 0, 0),
            )
        ],
        core_axis_name=("c", "s"),
        dimension_semantics=(pltpu.PARALLEL,),
    )(idx_hbm, out_hbm)


def kernel(inputs, table):
    batch, seq = inputs.shape
    n = batch * seq
    n_bh = batch // _W
    idx = inputs.astype(jnp.int32).T.reshape(1, n)
    num_windows = n // _W

    @pl.kernel(
        out_type=jax.ShapeDtypeStruct((seq, 8, n_bh, 8, _W), table.dtype),
        mesh=plsc.VectorSubcoreMesh(core_axis_name="c", subcore_axis_name="s"),
        compiler_params=pltpu.CompilerParams(
            use_tc_tiling_on_sc=False, needs_layout_passes=False),
        scratch_types=[
            pltpu.VMEM((_W, _D), jnp.float32),
            pltpu.VMEM((_D // _L, _W, 17), jnp.float32),
        ],
    )
    def emb(table_hbm, idx_hbm, out_hbm, rows_v, st_v):
        _emb_pipeline(table_hbm, idx_hbm, out_hbm, rows_v, st_v,
                      num_windows=num_windows, n_bh=n_bh)

    out5d = emb(table, idx)
    return out5d.transpose(2, 4, 0, 1, 3).reshape(batch, seq, _D)
